# Initial kernel scaffold; baseline (speedup 1.0000x reference)
import jax
import jax.numpy as jnp
from jax.experimental import pallas as pl


def kernel(center, outside, negative, emb_v, emb_u):
    raise NotImplementedError("write your pallas kernel here")


# speedup vs baseline: 4.3765x; 4.3765x over previous
import functools

import jax
import jax.numpy as jnp
from jax import lax
from jax.experimental import pallas as pl
from jax.experimental.pallas import tpu as pltpu
from jax.experimental.pallas import tpu_sc as plsc

NC = 2
NS = 16
NW = NC * NS
CH = 128
KBUF = 4


def _sc_gather(B, K, D, center2d, outside2d, negative2d, emb_v, emb_u):
    RPW = B // NW
    NCH_C = RPW // CH
    NCH_N = RPW * K // CH
    NR = NCH_N // KBUF
    assert NCH_C == KBUF and NR >= 2 and NR % 2 == 0

    mesh = plsc.VectorSubcoreMesh(core_axis_name="c", subcore_axis_name="s")

    def body(center_r, outside_r, negative_r, ev, eu, out_v, out_o, out_n,
             idx_c, idx_o, idx_n, bufs, gsem, osem):
        wid = lax.axis_index("s") * NC + lax.axis_index("c")
        vbase = wid * RPW
        nbase = wid * RPW * K

        pltpu.sync_copy(center_r.at[pl.ds(wid * NCH_C, NCH_C)], idx_c)
        pltpu.sync_copy(outside_r.at[pl.ds(wid * NCH_C, NCH_C)], idx_o)
        pltpu.sync_copy(negative_r.at[pl.ds(wid * NCH_N, NCH_N)], idx_n)

        def fire_g(table, idxrow, s, b):
            pltpu.async_copy(table.at[idxrow], bufs.at[s, b], gsem)

        def drain_g(table, idxrow, s, b):
            pltpu.make_async_copy(table.at[idxrow], bufs.at[s, b], gsem).wait()

        def fire_o(s, b, out, row):
            pltpu.async_copy(bufs.at[s, b], out.at[pl.ds(row, CH)], osem)

        def drain_o(s, b, out, row):
            pltpu.make_async_copy(bufs.at[s, b], out.at[pl.ds(row, CH)],
                                  osem).wait()

        for b in range(KBUF):
            fire_g(ev, idx_c.at[b], 0, b)
        for b in range(KBUF):
            drain_g(ev, idx_c.at[b], 0, b)
        for b in range(KBUF):
            fire_o(0, b, out_v, vbase + b * CH)

        for b in range(KBUF):
            fire_g(eu, idx_o.at[b], 1, b)
        for b in range(KBUF):
            drain_g(eu, idx_o.at[b], 1, b)
        for b in range(KBUF):
            drain_o(0, b, out_v, vbase + b * CH)
        for b in range(KBUF):
            fire_o(1, b, out_o, vbase + b * CH)

        def n_fire_g(r, s):
            for b in range(KBUF):
                fire_g(eu, idx_n.at[r * KBUF + b], s, b)

        def n_drain_g(r, s):
            for b in range(KBUF):
                drain_g(eu, idx_n.at[r * KBUF + b], s, b)

        def n_fire_o(r, s):
            for b in range(KBUF):
                fire_o(s, b, out_n, nbase + (r * KBUF + b) * CH)

        def n_drain_o(r, s):
            for b in range(KBUF):
                drain_o(s, b, out_n, nbase + (r * KBUF + b) * CH)

        n_fire_g(0, 0)
        n_drain_g(0, 0)
        for b in range(KBUF):
            drain_o(1, b, out_o, vbase + b * CH)
        n_fire_o(0, 0)
        n_fire_g(1, 1)

        def pair(i, carry):
            r1 = 2 * i + 1
            n_drain_g(r1, 1)
            n_drain_o(r1 - 1, 0)
            n_fire_o(r1, 1)
            n_fire_g(r1 + 1, 0)
            r2 = 2 * i + 2
            n_drain_g(r2, 0)
            n_drain_o(r2 - 1, 1)
            n_fire_o(r2, 0)
            n_fire_g(r2 + 1, 1)
            return carry

        lax.fori_loop(0, (NR - 2) // 2, pair, 0)

        rl = NR - 1
        n_drain_g(rl, 1)
        n_drain_o(rl - 1, 0)
        n_fire_o(rl, 1)
        n_drain_o(rl, 1)

    f32 = jnp.float32
    run = pl.kernel(
        body,
        out_type=[
            jax.ShapeDtypeStruct((B, D), f32),
            jax.ShapeDtypeStruct((B, D), f32),
            jax.ShapeDtypeStruct((B * K, D), f32),
        ],
        mesh=mesh,
        compiler_params=pltpu.CompilerParams(use_tc_tiling_on_sc=False),
        scratch_types=[
            pltpu.VMEM((NCH_C, CH), jnp.int32),
            pltpu.VMEM((NCH_C, CH), jnp.int32),
            pltpu.VMEM((NCH_N, CH), jnp.int32),
            pltpu.VMEM((2, KBUF, CH, D), f32),
            pltpu.SemaphoreType.DMA,
            pltpu.SemaphoreType.DMA,
        ],
    )
    return run(center2d, outside2d, negative2d, emb_v, emb_u)


def _tc_loss(B, K, D, v_c, u_o, u_n):
    BV = 512
    G = B // BV

    def body(v_ref, o_ref, n_ref, acc_ref):
        i = pl.program_id(0)
        v = v_ref[...]
        pos = jnp.sum(v * o_ref[...], axis=1)
        vb = jnp.reshape(jnp.broadcast_to(v[:, None, :], (BV, K, D)),
                         (BV * K, D))
        neg = jnp.sum(n_ref[...] * vb, axis=1)
        def logsig(x):
            return -jnp.log1p(jnp.exp(-x))

        pos_l = logsig(jnp.clip(pos, -10.0, 10.0))
        neg_l = logsig(jnp.clip(-neg, -10.0, 10.0))
        part = jnp.reshape(jnp.sum(pos_l) + jnp.sum(neg_l), (1, 1))

        @pl.when(i == 0)
        def _():
            acc_ref[...] = jnp.zeros_like(acc_ref)

        acc_ref[...] += -part

    out = pl.pallas_call(
        body,
        grid=(G,),
        in_specs=[
            pl.BlockSpec((BV, D), lambda i: (i, 0)),
            pl.BlockSpec((BV, D), lambda i: (i, 0)),
            pl.BlockSpec((BV * K, D), lambda i: (i, 0)),
        ],
        out_specs=pl.BlockSpec((1, 1), lambda i: (0, 0)),
        out_shape=jax.ShapeDtypeStruct((1, 1), jnp.float32),
        compiler_params=pltpu.CompilerParams(
            dimension_semantics=("arbitrary",)),
    )(v_c, u_o, u_n)
    return out[0, 0]


def kernel(center, outside, negative, emb_v, emb_u):
    B, = center.shape
    K = negative.shape[1]
    D = emb_v.shape[1]
    center2d = center.reshape(B // CH, CH)
    outside2d = outside.reshape(B // CH, CH)
    negative2d = negative.reshape(B * K // CH, CH)
    v_c, u_o, u_n = _sc_gather(B, K, D, center2d, outside2d, negative2d,
                               emb_v, emb_u)
    return _tc_loss(B, K, D, v_c, u_o, u_n)
